# baseline (speedup 1.0000x reference)
import functools

import jax
import jax.numpy as jnp
import numpy as np
from jax import lax
from jax.experimental import pallas as pl
from jax.experimental.pallas import tpu as pltpu
from jax.experimental.pallas import tpu_sc as plsc

_ORDER_CACHE = {}


def _perm_order(n):
    if n not in _ORDER_CACHE:
        try:
            with jax.ensure_compile_time_eval():
                order = jax.random.permutation(jax.random.key(42), n)
            _ORDER_CACHE[n] = np.asarray(order, dtype=np.int32)
        except Exception:
            _ORDER_CACHE[n] = None
    const = _ORDER_CACHE[n]
    if const is not None:
        return jnp.asarray(const)
    return jax.random.permutation(jax.random.key(42), n).astype(jnp.int32)


@functools.lru_cache(maxsize=None)
def _build_gather(rows, d, chunk=8, nbuf=4):
    info = plsc.get_sparse_core_info()
    nc, ns = info.num_cores, info.num_subcores
    nw = nc * ns
    assert rows % nw == 0
    b_per_w = rows // nw
    assert b_per_w % chunk == 0
    nchunks = b_per_w // chunk
    assert nchunks % nbuf == 0
    mesh = plsc.VectorSubcoreMesh(core_axis_name="c", subcore_axis_name="s")

    @functools.partial(
        pl.kernel,
        mesh=mesh,
        out_type=jax.ShapeDtypeStruct((rows, d), jnp.float32),
        scratch_types=[
            pltpu.VMEM((b_per_w,), jnp.int32),
            pltpu.VMEM((nbuf * chunk, d), jnp.float32),
        ] + [pltpu.SemaphoreType.DMA] * (2 * nbuf),
    )
    def gather_kernel(x_hbm, idx_hbm, out_hbm, idx_v, buf, *sems):
        wid = lax.axis_index("s") * nc + lax.axis_index("c")
        base = wid * b_per_w
        pltpu.sync_copy(idx_hbm.at[pl.ds(base, b_per_w)], idx_v)
        gsem = sems[:nbuf]
        osem = sems[nbuf:]

        def bslice(b):
            return buf.at[pl.ds(b * chunk, chunk)]

        def gather_copy(c, b):
            return pltpu.make_async_copy(
                x_hbm.at[idx_v.at[pl.ds(c * chunk, chunk)]],
                bslice(b), gsem[b])

        def out_copy(c, b):
            return pltpu.make_async_copy(
                bslice(b), out_hbm.at[pl.ds(base + c * chunk, chunk)],
                osem[b])

        for k in range(nbuf - 1):
            gather_copy(k, k).start()

        @pl.loop(0, nchunks, step=nbuf)
        def _group(c0):
            for j in range(nbuf):
                s = c0 + j
                b = j
                pb = (j - 1) % nbuf
                gather_copy(s, b).wait()
                out_copy(s, b).start()

                @pl.when(s >= 1)
                def _free_prev():
                    out_copy(s - 1, pb).wait()

                @pl.when(s + nbuf - 1 < nchunks)
                def _refill():
                    gather_copy(s + nbuf - 1, pb).start()

        out_copy(nchunks - 1, (nchunks - 1) % nbuf).wait()

    return gather_kernel


@functools.lru_cache(maxsize=None)
def _build_tc_gather(shape, rows_out, k=8):
    n, s0, s1 = shape
    assert rows_out % k == 0
    grid = (rows_out // k,)

    def body(order_ref, *refs):
        del order_ref
        x_refs, o_ref = refs[:k], refs[k]
        for j in range(k):
            o_ref[pl.ds(j, 1)] = x_refs[j][...]

    def in_map(j):
        return lambda i, order: (order[i * k + j], 0, 0)

    del n

    return pl.pallas_call(
        body,
        grid_spec=pltpu.PrefetchScalarGridSpec(
            num_scalar_prefetch=1,
            grid=grid,
            in_specs=[pl.BlockSpec((1, s0, s1), in_map(j)) for j in range(k)],
            out_specs=pl.BlockSpec((k, s0, s1), lambda i, order: (i, 0, 0)),
        ),
        out_shape=jax.ShapeDtypeStruct((rows_out, s0, s1), jnp.float32),
    )


def kernel(x):
    n = x.shape[0]
    d = 1
    for s in x.shape[1:]:
        d *= s
    order = _perm_order(n)
    rows_tc = 6144 if n == 16384 else 0
    rows_sc = n - rows_tc
    sc_out = _build_gather(rows_sc, d)(x.reshape(n, d), order[rows_tc:])
    sc_out = sc_out.reshape((rows_sc,) + x.shape[1:])
    if rows_tc == 0:
        return sc_out
    tc_out = _build_tc_gather(x.shape, rows_tc)(order[:rows_tc], *([x] * 8))
    return jnp.concatenate([tc_out, sc_out], axis=0)


# speedup vs baseline: 1.0137x; 1.0137x over previous
import functools

import jax
import jax.numpy as jnp
import numpy as np
from jax import lax
from jax.experimental import pallas as pl
from jax.experimental.pallas import tpu as pltpu
from jax.experimental.pallas import tpu_sc as plsc

_ORDER_CACHE = {}


def _perm_order(n):
    if n not in _ORDER_CACHE:
        try:
            with jax.ensure_compile_time_eval():
                order = jax.random.permutation(jax.random.key(42), n)
            _ORDER_CACHE[n] = np.asarray(order, dtype=np.int32)
        except Exception:
            _ORDER_CACHE[n] = None
    const = _ORDER_CACHE[n]
    if const is not None:
        return jnp.asarray(const)
    return jax.random.permutation(jax.random.key(42), n).astype(jnp.int32)


@functools.lru_cache(maxsize=None)
def _build_gather(n, row0, d, chunk=8, nbuf=4):
    rows = n - row0
    info = plsc.get_sparse_core_info()
    nc, ns = info.num_cores, info.num_subcores
    nw = nc * ns
    assert rows % nw == 0
    b_per_w = rows // nw
    assert b_per_w % chunk == 0
    nchunks = b_per_w // chunk
    assert nchunks % nbuf == 0
    mesh = plsc.VectorSubcoreMesh(core_axis_name="c", subcore_axis_name="s")

    @functools.partial(
        pl.kernel,
        mesh=mesh,
        out_type=jax.ShapeDtypeStruct((n, d), jnp.float32),
        scratch_types=[
            pltpu.VMEM((b_per_w,), jnp.int32),
            pltpu.VMEM((nbuf * chunk, d), jnp.float32),
        ] + [pltpu.SemaphoreType.DMA] * (2 * nbuf),
    )
    def gather_kernel(x_hbm, idx_hbm, out_hbm, idx_v, buf, *sems):
        wid = lax.axis_index("s") * nc + lax.axis_index("c")
        base = wid * b_per_w
        obase = row0 + base
        pltpu.sync_copy(idx_hbm.at[pl.ds(base, b_per_w)], idx_v)
        gsem = sems[:nbuf]
        osem = sems[nbuf:]

        def bslice(b):
            return buf.at[pl.ds(b * chunk, chunk)]

        def gather_copy(c, b):
            return pltpu.make_async_copy(
                x_hbm.at[idx_v.at[pl.ds(c * chunk, chunk)]],
                bslice(b), gsem[b])

        def out_copy(c, b):
            return pltpu.make_async_copy(
                bslice(b), out_hbm.at[pl.ds(obase + c * chunk, chunk)],
                osem[b])

        for k in range(nbuf - 1):
            gather_copy(k, k).start()

        @pl.loop(0, nchunks, step=nbuf)
        def _group(c0):
            for j in range(nbuf):
                s = c0 + j
                b = j
                pb = (j - 1) % nbuf
                gather_copy(s, b).wait()
                out_copy(s, b).start()

                @pl.when(s >= 1)
                def _free_prev():
                    out_copy(s - 1, pb).wait()

                @pl.when(s + nbuf - 1 < nchunks)
                def _refill():
                    gather_copy(s + nbuf - 1, pb).start()

        out_copy(nchunks - 1, (nchunks - 1) % nbuf).wait()

    return gather_kernel


@functools.lru_cache(maxsize=None)
def _build_tc_gather(shape, rows_out, k=8):
    n, s0, s1 = shape
    assert rows_out % k == 0
    grid = (rows_out // k,)

    def body(order_ref, *refs):
        del order_ref
        x_refs, o_ref = refs[:k], refs[k]
        for j in range(k):
            o_ref[pl.ds(j, 1)] = x_refs[j][...]

    def in_map(j):
        return lambda i, order: (order[i * k + j], 0, 0)

    del n

    return pl.pallas_call(
        body,
        grid_spec=pltpu.PrefetchScalarGridSpec(
            num_scalar_prefetch=1,
            grid=grid,
            in_specs=[pl.BlockSpec((1, s0, s1), in_map(j)) for j in range(k)],
            out_specs=pl.BlockSpec((k, s0, s1), lambda i, order: (i, 0, 0)),
        ),
        out_shape=jax.ShapeDtypeStruct((rows_out, s0, s1), jnp.float32),
    )


def kernel(x):
    n = x.shape[0]
    d = 1
    for s in x.shape[1:]:
        d *= s
    order = _perm_order(n)
    rows_tc = 6144 if n == 16384 else 0
    sc_out = _build_gather(n, rows_tc, d)(x.reshape(n, d), order[rows_tc:])
    sc_out = sc_out.reshape(x.shape)
    if rows_tc == 0:
        return sc_out
    tc_out = _build_tc_gather(x.shape, rows_tc)(order[:rows_tc], *([x] * 8))
    return lax.dynamic_update_slice(sc_out, tc_out, (0, 0, 0))


# speedup vs baseline: 1.7975x; 1.7732x over previous
import functools

import jax
import jax.numpy as jnp
import numpy as np
from jax import lax
from jax.experimental import pallas as pl
from jax.experimental.pallas import tpu as pltpu
from jax.experimental.pallas import tpu_sc as plsc

_ORDER_CACHE = {}


def _perm_order(n):
    if n not in _ORDER_CACHE:
        try:
            with jax.ensure_compile_time_eval():
                order = jax.random.permutation(jax.random.key(42), n)
            _ORDER_CACHE[n] = np.asarray(order, dtype=np.int32)
        except Exception:
            _ORDER_CACHE[n] = None
    const = _ORDER_CACHE[n]
    if const is not None:
        return jnp.asarray(const)
    return jax.random.permutation(jax.random.key(42), n).astype(jnp.int32)


@functools.lru_cache(maxsize=None)
def _build_gather(n, row0, d, chunk=8, nbuf=4):
    rows = n - row0
    info = plsc.get_sparse_core_info()
    nc, ns = info.num_cores, info.num_subcores
    nw = nc * ns
    assert rows % nw == 0
    b_per_w = rows // nw
    assert b_per_w % chunk == 0
    nchunks = b_per_w // chunk
    assert nchunks % nbuf == 0
    mesh = plsc.VectorSubcoreMesh(core_axis_name="c", subcore_axis_name="s")

    @functools.partial(
        pl.kernel,
        mesh=mesh,
        out_type=jax.ShapeDtypeStruct((n, d), jnp.float32),
        scratch_types=[
            pltpu.VMEM((b_per_w,), jnp.int32),
            pltpu.VMEM((nbuf * chunk, d), jnp.float32),
        ] + [pltpu.SemaphoreType.DMA] * (2 * nbuf),
    )
    def gather_kernel(x_hbm, idx_hbm, out_hbm, idx_v, buf, *sems):
        wid = lax.axis_index("s") * nc + lax.axis_index("c")
        base = wid * b_per_w
        obase = row0 + base
        pltpu.sync_copy(idx_hbm.at[pl.ds(base, b_per_w)], idx_v)
        gsem = sems[:nbuf]
        osem = sems[nbuf:]

        def bslice(b):
            return buf.at[pl.ds(b * chunk, chunk)]

        def gather_copy(c, b):
            return pltpu.make_async_copy(
                x_hbm.at[idx_v.at[pl.ds(c * chunk, chunk)]],
                bslice(b), gsem[b])

        def out_copy(c, b):
            return pltpu.make_async_copy(
                bslice(b), out_hbm.at[pl.ds(obase + c * chunk, chunk)],
                osem[b])

        for k in range(nbuf - 1):
            gather_copy(k, k).start()

        @pl.loop(0, nchunks, step=nbuf)
        def _group(c0):
            for j in range(nbuf):
                s = c0 + j
                b = j
                pb = (j - 1) % nbuf
                gather_copy(s, b).wait()
                out_copy(s, b).start()

                @pl.when(s >= 1)
                def _free_prev():
                    out_copy(s - 1, pb).wait()

                @pl.when(s + nbuf - 1 < nchunks)
                def _refill():
                    gather_copy(s + nbuf - 1, pb).start()

        out_copy(nchunks - 1, (nchunks - 1) % nbuf).wait()

    return gather_kernel


def kernel(x):
    n = x.shape[0]
    d = 1
    for s in x.shape[1:]:
        d *= s
    order = _perm_order(n)
    out = _build_gather(n, 0, d)(x.reshape(n, d), order)
    return out.reshape(x.shape)
